# baseline (speedup 1.0000x reference)
import functools

import jax
import jax.numpy as jnp
from jax import lax
from jax.experimental import pallas as pl
from jax.experimental.pallas import tpu as pltpu
from jax.experimental.pallas import tpu_sc as plsc

_K = 10
_GAMMA = 1.0
_R = 512
_C = 512
_CSUB = 256
_RSUB = 64
_PADV = 3.0e5


def _sc_gather_pair(table1, table2, idx1, idx2):
    B = idx1.shape[0]
    D = table1.shape[1]
    info = plsc.get_sparse_core_info()
    nw = info.num_cores * info.num_subcores
    b_per_w = B // nw
    mesh = plsc.VectorSubcoreMesh(core_axis_name="c", subcore_axis_name="s")

    @functools.partial(
        pl.kernel,
        mesh=mesh,
        out_type=[
            jax.ShapeDtypeStruct((B, D), jnp.float32),
            jax.ShapeDtypeStruct((B, D), jnp.float32),
        ],
        scratch_types=[
            pltpu.VMEM((b_per_w,), jnp.int32),
            pltpu.VMEM((b_per_w, D), jnp.float32),
            pltpu.SemaphoreType.DMA,
        ],
    )
    def gather_k(t1_hbm, t2_hbm, i1_hbm, i2_hbm, o1_hbm, o2_hbm, idx_v, rows_v, sem):
        wid = lax.axis_index("s") * info.num_cores + lax.axis_index("c")
        base = wid * b_per_w
        pltpu.sync_copy(i1_hbm.at[pl.ds(base, b_per_w)], idx_v)
        pltpu.async_copy(t1_hbm.at[idx_v], rows_v, sem).wait()
        pltpu.sync_copy(rows_v, o1_hbm.at[pl.ds(base, b_per_w)])
        pltpu.sync_copy(i2_hbm.at[pl.ds(base, b_per_w)], idx_v)
        pltpu.async_copy(t2_hbm.at[idx_v], rows_v, sem).wait()
        pltpu.sync_copy(rows_v, o2_hbm.at[pl.ds(base, b_per_w)])

    return gather_k(table1, table2, idx1, idx2)


def _loss_body(anch_ref, tgt_ref, a1_ref, a2_ref, out_ref, cand_ref, *, ncb, nplanes):
    i = pl.program_id(0)
    j = pl.program_id(1)

    D = anch_ref.shape[1]
    a = anch_ref[...]
    sa = jnp.sum(a, axis=1, keepdims=True)
    lane_c = lax.broadcasted_iota(jnp.int32, (_R, _CSUB), 1).astype(jnp.float32)
    lane128 = lax.broadcasted_iota(jnp.int32, (_R, 128), 1)
    nch = _C // _CSUB
    big = jnp.float32(3.0e38)

    @pl.when(j == 0)
    def _init():
        for p in range(nplanes):
            cand_ref[p] = jnp.full((_R, 128), big, jnp.float32)

    bidx = j * nch
    pidx = (j * nch) // 8
    for ci in range(nch):
        c0 = ci * _CSUB
        bT = tgt_ref[0, :, c0:c0 + _CSUB]
        sb = jnp.sum(bT, axis=0, keepdims=True)
        lane_base = ((bidx + ci) % 8) * 16
        macc = jnp.zeros((_R, _CSUB), jnp.float32)
        for g in range(D // 4):
            d0 = 4 * g
            t01 = (jnp.minimum(a[:, d0:d0 + 1], bT[d0:d0 + 1, :]) +
                   jnp.minimum(a[:, d0 + 1:d0 + 2], bT[d0 + 1:d0 + 2, :]))
            t23 = (jnp.minimum(a[:, d0 + 2:d0 + 3], bT[d0 + 2:d0 + 3, :]) +
                   jnp.minimum(a[:, d0 + 3:d0 + 4], bT[d0 + 3:d0 + 4, :]))
            macc = macc + (t01 + t23)
        fkey = jnp.floor((sa + sb - 2.0 * macc) * 16.0) * float(_CSUB) + lane_c

        plane = cand_ref[pidx]
        for k in range(_K):
            mk = jnp.min(fkey, axis=1, keepdims=True)
            fkey = jnp.where(fkey == mk, big, fkey)
            plane = jnp.where(lane128 == lane_base + k, mk, plane)
        cand_ref[pidx] = plane

    @pl.when(j == ncb - 1)
    def _final():
        a1 = a1_ref[...]
        a2 = a2_ref[...]
        dm = jnp.sum(jnp.abs(a1 - a2), axis=1, keepdims=True) + _GAMMA
        k2 = jnp.concatenate([cand_ref[t] for t in range(nplanes)], axis=1)
        terms = jnp.zeros((_R, 1), jnp.float32)
        for _ in range(_K):
            mk = jnp.min(k2, axis=1, keepdims=True)
            k2 = jnp.where(k2 == mk, big, k2)
            v = jnp.floor(mk * (1.0 / _CSUB)) * (1.0 / 16.0) + (1.0 / 32.0)
            terms = terms + jnp.maximum(dm - v, 0.0)
        contrib = jnp.sum(terms, keepdims=True)
        z = jnp.zeros((8, 128), jnp.float32)
        r8 = lax.broadcasted_iota(jnp.int32, (8, 128), 0)
        l8 = lax.broadcasted_iota(jnp.int32, (8, 128), 1)
        out_ref[...] = jnp.where((r8 == 0) & (l8 == 0), contrib, z)


def _tc_loss(a1v, a2v, out1, out2):
    T, D = a1v.shape
    N = out1.shape[0]
    npad = ((N + _C - 1) // _C) * _C
    pad = jnp.full((npad - N, D), _PADV, jnp.float32)
    t1 = jnp.concatenate([out2, pad], 0).T
    t2 = jnp.concatenate([out1, pad], 0).T
    tgt = jnp.stack([t1, t2])
    anchors = jnp.concatenate([a1v, a2v], axis=0)
    per_side = T // _R
    ncb = npad // _C
    nplanes = ncb * (_C // _CSUB) * 16 // 128
    grid = (2 * T // _R, ncb)
    total = pl.pallas_call(
        functools.partial(_loss_body, ncb=ncb, nplanes=nplanes),
        grid=grid,
        in_specs=[
            pl.BlockSpec((_R, D), lambda i, j: (i, 0)),
            pl.BlockSpec((1, D, _C), lambda i, j: (i // per_side, 0, j)),
            pl.BlockSpec((_R, D), lambda i, j: (i % per_side, 0)),
            pl.BlockSpec((_R, D), lambda i, j: (i % per_side, 0)),
        ],
        out_specs=pl.BlockSpec((8, 128), lambda i, j: (i, 0)),
        out_shape=jax.ShapeDtypeStruct((grid[0] * 8, 128), jnp.float32),
        scratch_shapes=[pltpu.VMEM((nplanes, _R, 128), jnp.float32)],
        compiler_params=pltpu.CompilerParams(
            dimension_semantics=("parallel", "arbitrary")),
    )(anchors, tgt, a1v, a2v)
    return jnp.sum(total) / (T * _K)


def kernel(out1, out2, anchor1_org, anchor2_org):
    a1v, a2v = _sc_gather_pair(out1, out2, anchor1_org, anchor2_org)
    return _tc_loss(a1v, a2v, out1, out2)


# speedup vs baseline: 1.1086x; 1.1086x over previous
import functools

import jax
import jax.numpy as jnp
from jax import lax
from jax.experimental import pallas as pl
from jax.experimental.pallas import tpu as pltpu
from jax.experimental.pallas import tpu_sc as plsc

_K = 10
_GAMMA = 1.0
_R = 512
_C = 512
_CSUB = 512
_RSUB = 64
_PADV = 3.0e5


def _sc_gather_pair(table1, table2, idx1, idx2):
    B = idx1.shape[0]
    D = table1.shape[1]
    info = plsc.get_sparse_core_info()
    nw = info.num_cores * info.num_subcores
    b_per_w = B // nw
    mesh = plsc.VectorSubcoreMesh(core_axis_name="c", subcore_axis_name="s")

    @functools.partial(
        pl.kernel,
        mesh=mesh,
        out_type=[
            jax.ShapeDtypeStruct((B, D), jnp.float32),
            jax.ShapeDtypeStruct((B, D), jnp.float32),
        ],
        scratch_types=[
            pltpu.VMEM((b_per_w,), jnp.int32),
            pltpu.VMEM((b_per_w, D), jnp.float32),
            pltpu.SemaphoreType.DMA,
        ],
    )
    def gather_k(t1_hbm, t2_hbm, i1_hbm, i2_hbm, o1_hbm, o2_hbm, idx_v, rows_v, sem):
        wid = lax.axis_index("s") * info.num_cores + lax.axis_index("c")
        base = wid * b_per_w
        pltpu.sync_copy(i1_hbm.at[pl.ds(base, b_per_w)], idx_v)
        pltpu.async_copy(t1_hbm.at[idx_v], rows_v, sem).wait()
        pltpu.sync_copy(rows_v, o1_hbm.at[pl.ds(base, b_per_w)])
        pltpu.sync_copy(i2_hbm.at[pl.ds(base, b_per_w)], idx_v)
        pltpu.async_copy(t2_hbm.at[idx_v], rows_v, sem).wait()
        pltpu.sync_copy(rows_v, o2_hbm.at[pl.ds(base, b_per_w)])

    return gather_k(table1, table2, idx1, idx2)


def _loss_body(anch_ref, tgt_ref, a1_ref, a2_ref, out_ref, cand_ref, *, ncb, nplanes):
    i = pl.program_id(0)
    j = pl.program_id(1)

    D = anch_ref.shape[1]
    a = anch_ref[...]
    sa = jnp.sum(a, axis=1, keepdims=True)
    lane_c = lax.broadcasted_iota(jnp.int32, (_R, _CSUB), 1).astype(jnp.float32)
    lane128 = lax.broadcasted_iota(jnp.int32, (_R, 128), 1)
    nch = _C // _CSUB
    big = jnp.float32(3.0e38)

    @pl.when(j == 0)
    def _init():
        for p in range(nplanes):
            cand_ref[p] = jnp.full((_R, 128), big, jnp.float32)

    bidx = j * nch
    pidx = (j * nch) // 8
    for ci in range(nch):
        c0 = ci * _CSUB
        bT = tgt_ref[0, :, c0:c0 + _CSUB]
        sb = jnp.sum(bT, axis=0, keepdims=True)
        lane_base = ((bidx + ci) % 8) * 16
        macc = jnp.zeros((_R, _CSUB), jnp.float32)
        for d in range(D):
            macc = macc + jnp.minimum(a[:, d:d + 1], bT[d:d + 1, :])
        fkey = jnp.floor((sa + sb - 2.0 * macc) * 16.0) * float(_CSUB) + lane_c

        plane = cand_ref[pidx]
        for k in range(_K):
            mk = jnp.min(fkey, axis=1, keepdims=True)
            fkey = jnp.where(fkey == mk, big, fkey)
            plane = jnp.where(lane128 == lane_base + k, mk, plane)
        cand_ref[pidx] = plane

    @pl.when(j == ncb - 1)
    def _final():
        a1 = a1_ref[...]
        a2 = a2_ref[...]
        dm = jnp.sum(jnp.abs(a1 - a2), axis=1, keepdims=True) + _GAMMA
        k2 = jnp.concatenate([cand_ref[t] for t in range(nplanes)], axis=1)
        terms = jnp.zeros((_R, 1), jnp.float32)
        for _ in range(_K):
            mk = jnp.min(k2, axis=1, keepdims=True)
            k2 = jnp.where(k2 == mk, big, k2)
            v = jnp.floor(mk * (1.0 / _CSUB)) * (1.0 / 16.0) + (1.0 / 32.0)
            terms = terms + jnp.maximum(dm - v, 0.0)
        contrib = jnp.sum(terms, keepdims=True)
        z = jnp.zeros((8, 128), jnp.float32)
        r8 = lax.broadcasted_iota(jnp.int32, (8, 128), 0)
        l8 = lax.broadcasted_iota(jnp.int32, (8, 128), 1)
        out_ref[...] = jnp.where((r8 == 0) & (l8 == 0), contrib, z)


def _tc_loss(a1v, a2v, out1, out2):
    T, D = a1v.shape
    N = out1.shape[0]
    npad = ((N + _C - 1) // _C) * _C
    pad = jnp.full((npad - N, D), _PADV, jnp.float32)
    t1 = jnp.concatenate([out2, pad], 0).T
    t2 = jnp.concatenate([out1, pad], 0).T
    tgt = jnp.stack([t1, t2])
    anchors = jnp.concatenate([a1v, a2v], axis=0)
    per_side = T // _R
    ncb = npad // _C
    nplanes = (ncb * (_C // _CSUB) * 16 + 127) // 128
    grid = (2 * T // _R, ncb)
    total = pl.pallas_call(
        functools.partial(_loss_body, ncb=ncb, nplanes=nplanes),
        grid=grid,
        in_specs=[
            pl.BlockSpec((_R, D), lambda i, j: (i, 0)),
            pl.BlockSpec((1, D, _C), lambda i, j: (i // per_side, 0, j)),
            pl.BlockSpec((_R, D), lambda i, j: (i % per_side, 0)),
            pl.BlockSpec((_R, D), lambda i, j: (i % per_side, 0)),
        ],
        out_specs=pl.BlockSpec((8, 128), lambda i, j: (i, 0)),
        out_shape=jax.ShapeDtypeStruct((grid[0] * 8, 128), jnp.float32),
        scratch_shapes=[pltpu.VMEM((nplanes, _R, 128), jnp.float32)],
    )(anchors, tgt, a1v, a2v)
    return jnp.sum(total) / (T * _K)


def kernel(out1, out2, anchor1_org, anchor2_org):
    a1v, a2v = _sc_gather_pair(out1, out2, anchor1_org, anchor2_org)
    return _tc_loss(a1v, a2v, out1, out2)


# speedup vs baseline: 1.1086x; 1.0000x over previous
import functools

import jax
import jax.numpy as jnp
from jax import lax
from jax.experimental import pallas as pl
from jax.experimental.pallas import tpu as pltpu
from jax.experimental.pallas import tpu_sc as plsc

_K = 10
_GAMMA = 1.0
_R = 512
_C = 512
_CSUB = 512
_PADV = 3.0e5


def _sc_gather_pair(table1, table2, idx1, idx2):
    B = idx1.shape[0]
    D = table1.shape[1]
    info = plsc.get_sparse_core_info()
    nw = info.num_cores * info.num_subcores
    b_per_w = B // nw
    mesh = plsc.VectorSubcoreMesh(core_axis_name="c", subcore_axis_name="s")

    @functools.partial(
        pl.kernel,
        mesh=mesh,
        out_type=[
            jax.ShapeDtypeStruct((B, D), jnp.float32),
            jax.ShapeDtypeStruct((B, D), jnp.float32),
        ],
        scratch_types=[
            pltpu.VMEM((b_per_w,), jnp.int32),
            pltpu.VMEM((b_per_w, D), jnp.float32),
            pltpu.SemaphoreType.DMA,
        ],
    )
    def gather_k(t1_hbm, t2_hbm, i1_hbm, i2_hbm, o1_hbm, o2_hbm, idx_v, rows_v, sem):
        wid = lax.axis_index("s") * info.num_cores + lax.axis_index("c")
        base = wid * b_per_w
        pltpu.sync_copy(i1_hbm.at[pl.ds(base, b_per_w)], idx_v)
        pltpu.async_copy(t1_hbm.at[idx_v], rows_v, sem).wait()
        pltpu.sync_copy(rows_v, o1_hbm.at[pl.ds(base, b_per_w)])
        pltpu.sync_copy(i2_hbm.at[pl.ds(base, b_per_w)], idx_v)
        pltpu.async_copy(t2_hbm.at[idx_v], rows_v, sem).wait()
        pltpu.sync_copy(rows_v, o2_hbm.at[pl.ds(base, b_per_w)])

    return gather_k(table1, table2, idx1, idx2)


def _loss_body(anch_ref, tgt_ref, a1_ref, a2_ref, out_ref, cand_ref, *, ncb, nplanes):
    j = pl.program_id(1)

    D = anch_ref.shape[1]
    a = anch_ref[...]
    sa = jnp.sum(a, axis=1, keepdims=True)
    lane_c = lax.broadcasted_iota(jnp.int32, (_R, _CSUB), 1).astype(jnp.float32)
    lane128 = lax.broadcasted_iota(jnp.int32, (_R, 128), 1)
    nch = _C // _CSUB
    big = jnp.float32(3.0e38)

    @pl.when(j == 0)
    def _init():
        for p in range(nplanes):
            cand_ref[p] = jnp.full((_R, 128), big, jnp.float32)

    bidx = j * nch
    pidx = (j * nch) // 8
    for ci in range(nch):
        c0 = ci * _CSUB
        bT = tgt_ref[0, :, c0:c0 + _CSUB]
        sb = jnp.sum(bT, axis=0, keepdims=True)
        lane_base = ((bidx + ci) % 8) * 16
        macc = jnp.zeros((_R, _CSUB), jnp.float32)
        for d in range(D):
            macc = macc + jnp.minimum(a[:, d:d + 1], bT[d:d + 1, :])
        fkey = jnp.floor((sa + sb - 2.0 * macc) * 16.0) * float(_CSUB) + lane_c

        plane = cand_ref[pidx]
        for k in range(_K):
            mk = jnp.min(fkey, axis=1, keepdims=True)
            fkey = jnp.where(fkey == mk, big, fkey)
            plane = jnp.where(lane128 == lane_base + k, mk, plane)
        cand_ref[pidx] = plane

    @pl.when(j == ncb - 1)
    def _final():
        a1 = a1_ref[...]
        a2 = a2_ref[...]
        dm = jnp.sum(jnp.abs(a1 - a2), axis=1, keepdims=True) + _GAMMA
        k2 = jnp.concatenate([cand_ref[t] for t in range(nplanes)], axis=1)
        terms = jnp.zeros((_R, 1), jnp.float32)
        for _ in range(_K):
            mk = jnp.min(k2, axis=1, keepdims=True)
            k2 = jnp.where(k2 == mk, big, k2)
            v = jnp.floor(mk * (1.0 / _CSUB)) * (1.0 / 16.0) + (1.0 / 32.0)
            terms = terms + jnp.maximum(dm - v, 0.0)
        contrib = jnp.sum(terms, keepdims=True)
        z = jnp.zeros((8, 128), jnp.float32)
        r8 = lax.broadcasted_iota(jnp.int32, (8, 128), 0)
        l8 = lax.broadcasted_iota(jnp.int32, (8, 128), 1)
        out_ref[...] = jnp.where((r8 == 0) & (l8 == 0), contrib, z)


def _tc_loss(a1v, a2v, out1, out2):
    T, D = a1v.shape
    N = out1.shape[0]
    npad = ((N + _C - 1) // _C) * _C
    pad = jnp.full((npad - N, D), _PADV, jnp.float32)
    t1 = jnp.concatenate([out2, pad], 0).T
    t2 = jnp.concatenate([out1, pad], 0).T
    tgt = jnp.stack([t1, t2])
    anchors = jnp.concatenate([a1v, a2v], axis=0)
    per_side = T // _R
    ncb = npad // _C
    nplanes = (ncb * (_C // _CSUB) * 16 + 127) // 128
    grid = (2 * T // _R, ncb)
    total = pl.pallas_call(
        functools.partial(_loss_body, ncb=ncb, nplanes=nplanes),
        grid=grid,
        in_specs=[
            pl.BlockSpec((_R, D), lambda i, j: (i, 0)),
            pl.BlockSpec((1, D, _C), lambda i, j: (i // per_side, 0, j)),
            pl.BlockSpec((_R, D), lambda i, j: (i % per_side, 0)),
            pl.BlockSpec((_R, D), lambda i, j: (i % per_side, 0)),
        ],
        out_specs=pl.BlockSpec((8, 128), lambda i, j: (i, 0)),
        out_shape=jax.ShapeDtypeStruct((grid[0] * 8, 128), jnp.float32),
        scratch_shapes=[pltpu.VMEM((nplanes, _R, 128), jnp.float32)],
    )(anchors, tgt, a1v, a2v)
    return jnp.sum(total) / (T * _K)


def kernel(out1, out2, anchor1_org, anchor2_org):
    a1v, a2v = _sc_gather_pair(out1, out2, anchor1_org, anchor2_org)
    return _tc_loss(a1v, a2v, out1, out2)
